# baseline (speedup 1.0000x reference)
import jax
import jax.numpy as jnp
from jax.experimental import pallas as pl

BS = 8
NC = 10
T = 32
F = 2048
H = 2048
MEM = 60
NVID = 2 * BS
RPV = NC * T
ROWS = NVID * RPV
K = T // 16 + 1


def _fused_kernel(x_ref, mask_ref, wa1_ref, ba1_ref, wa2_ref, ba2_ref, w1_ref,
                  b1_ref, w2_ref, b2_ref, w3_ref, b3_ref, mem_ref,
                  scores_out_ref, ssel_ref, fs_ref, sacc_ref, cacc_ref,
                  u_ref):
    i = pl.program_id(0)
    x = x_ref[...]
    xb = x.astype(jnp.bfloat16)
    h = jnp.maximum(
        jax.lax.dot(xb, wa1_ref[...], preferred_element_type=jnp.float32)
        + ba1_ref[...], 0.0)
    out = x + jax.lax.dot(h.astype(jnp.bfloat16), wa2_ref[...],
                          preferred_element_type=jnp.float32) + ba2_ref[...]
    sqn = jnp.sum(out * out, axis=1, keepdims=True)
    fmag = jnp.sqrt(sqn)
    s = jnp.maximum(
        jax.lax.dot(out.astype(jnp.bfloat16), w1_ref[...],
                    preferred_element_type=jnp.float32) + b1_ref[...], 0.0)
    s = jnp.maximum(
        jax.lax.dot(s.astype(jnp.bfloat16), w2_ref[...],
                    preferred_element_type=jnp.float32) + b2_ref[...], 0.0)
    sc = jax.nn.sigmoid(
        jax.lax.dot(s, w3_ref[...], preferred_element_type=jnp.float32)
        + b3_ref[...])

    r_iota = jax.lax.broadcasted_iota(jnp.int32, (RPV, T), 0)
    t_iota = jax.lax.broadcasted_iota(jnp.int32, (RPV, T), 1)
    wsel_mat = jnp.where(r_iota % T == t_iota, 1.0 / NC, 0.0)
    cdims = (((0,), (0,)), ((), ()))
    score_mean = jax.lax.dot_general(sc, wsel_mat, cdims,
                                     precision=jax.lax.Precision.HIGHEST,
                                     preferred_element_type=jnp.float32)
    fmag_mean = jax.lax.dot_general(fmag, wsel_mat, cdims,
                                    precision=jax.lax.Precision.HIGHEST,
                                    preferred_element_type=jnp.float32)
    scores_out_ref[0] = score_mean

    mask_mean = jnp.mean(mask_ref[0], axis=0, keepdims=True)
    drop = fmag_mean * mask_mean

    iota = jax.lax.broadcasted_iota(jnp.int32, (1, T), 1)
    v = drop
    wsel = jnp.zeros((1, T), jnp.float32)
    for _ in range(K):
        m = jnp.max(v, axis=1, keepdims=True)
        idx = jnp.min(jnp.where(v >= m, iota, T), axis=1, keepdims=True)
        hit = iota == idx
        wsel = wsel + jnp.where(hit, 1.0, 0.0)
        v = jnp.where(hit, -jnp.inf, v)

    ssel_ref[0] = jnp.sum(score_mean * wsel, axis=1, keepdims=True) / K

    tt_iota = jax.lax.broadcasted_iota(jnp.int32, (T, RPV), 0)
    tj_iota = jax.lax.broadcasted_iota(jnp.int32, (T, RPV), 1)
    tile_mat = jnp.where(tj_iota % T == tt_iota, 1.0, 0.0)
    wt_tiled = jax.lax.dot(wsel, tile_mat,
                           precision=jax.lax.Precision.HIGHEST,
                           preferred_element_type=jnp.float32)
    c_iota = jax.lax.broadcasted_iota(jnp.int32, (NC, RPV), 0)
    j_iota = jax.lax.broadcasted_iota(jnp.int32, (NC, RPV), 1)
    gmat = jnp.where(j_iota // T == c_iota, wt_tiled / K, 0.0)
    sel = jax.lax.dot(gmat, out, precision=jax.lax.Precision.HIGHEST,
                      preferred_element_type=jnp.float32)

    fs_ref[0] = jnp.sqrt(jnp.sum(sel * sel, axis=1)).reshape(1, NC)

    mem = mem_ref[...]
    logits = jax.lax.dot_general(sel, mem, (((1,), (1,)), ((), ())),
                                 preferred_element_type=jnp.float32)
    att = jax.nn.softmax(logits, axis=-1)
    recon = jax.lax.dot(att, mem, preferred_element_type=jnp.float32)
    diff = recon - sel
    s_contrib = jnp.sum(diff * diff).reshape(1, 1)
    c_contrib = jnp.sum(att * jnp.log(att + 1e-8)).reshape(1, 1)

    @pl.when(i == 0)
    def _():
        sacc_ref[...] = jnp.zeros((1, 1), jnp.float32)
        cacc_ref[...] = jnp.zeros((1, 1), jnp.float32)
        mn = mem / (jnp.sqrt(jnp.sum(mem * mem, axis=1, keepdims=True)) + 1e-8)
        g = jax.lax.dot_general(mn, mn, (((1,), (1,)), ((), ())),
                                preferred_element_type=jnp.float32)
        r = jax.lax.broadcasted_iota(jnp.int32, (MEM, MEM), 0)
        c = jax.lax.broadcasted_iota(jnp.int32, (MEM, MEM), 1)
        gd = g - jnp.where(r == c, 1.0, 0.0)
        u_ref[...] = (jnp.sum(gd * gd) / (MEM * MEM)).reshape(1, 1)

    sacc_ref[...] += s_contrib
    cacc_ref[...] += c_contrib

    @pl.when(i == NVID - 1)
    def _():
        sacc_ref[...] = sacc_ref[...] / (NVID * NC * F)
        cacc_ref[...] = -cacc_ref[...] / (NVID * NC)


@jax.jit
def kernel(inputs, mask, Wa1, ba1, Wa2, ba2, W1, b1, W2, b2, W3, b3, Mem):
    x = inputs.reshape(ROWS, F)
    mask3 = mask.reshape(NVID, NC, T)

    scores_out, ssel, fs, s_loss, c_loss, u_loss = pl.pallas_call(
        _fused_kernel,
        grid=(NVID,),
        in_specs=[
            pl.BlockSpec((RPV, F), lambda i: (i, 0)),
            pl.BlockSpec((1, NC, T), lambda i: (i, 0, 0)),
            pl.BlockSpec((F, H), lambda i: (0, 0)),
            pl.BlockSpec((1, H), lambda i: (0, 0)),
            pl.BlockSpec((H, F), lambda i: (0, 0)),
            pl.BlockSpec((1, F), lambda i: (0, 0)),
            pl.BlockSpec((F, 512), lambda i: (0, 0)),
            pl.BlockSpec((1, 512), lambda i: (0, 0)),
            pl.BlockSpec((512, 128), lambda i: (0, 0)),
            pl.BlockSpec((1, 128), lambda i: (0, 0)),
            pl.BlockSpec((128, 1), lambda i: (0, 0)),
            pl.BlockSpec((1, 1), lambda i: (0, 0)),
            pl.BlockSpec((MEM, F), lambda i: (0, 0)),
        ],
        out_specs=[
            pl.BlockSpec((1, 1, T), lambda i: (i, 0, 0)),
            pl.BlockSpec((1, 1, 1), lambda i: (i, 0, 0)),
            pl.BlockSpec((1, 1, NC), lambda i: (i, 0, 0)),
            pl.BlockSpec((1, 1), lambda i: (0, 0)),
            pl.BlockSpec((1, 1), lambda i: (0, 0)),
            pl.BlockSpec((1, 1), lambda i: (0, 0)),
        ],
        out_shape=[
            jax.ShapeDtypeStruct((NVID, 1, T), jnp.float32),
            jax.ShapeDtypeStruct((NVID, 1, 1), jnp.float32),
            jax.ShapeDtypeStruct((NVID, 1, NC), jnp.float32),
            jax.ShapeDtypeStruct((1, 1), jnp.float32),
            jax.ShapeDtypeStruct((1, 1), jnp.float32),
            jax.ShapeDtypeStruct((1, 1), jnp.float32),
        ],
    )(x, mask3, Wa1.astype(jnp.bfloat16), ba1.reshape(1, H),
      Wa2.astype(jnp.bfloat16), ba2.reshape(1, F), W1.astype(jnp.bfloat16),
      b1.reshape(1, 512), W2.astype(jnp.bfloat16), b2.reshape(1, 128), W3,
      b3.reshape(1, 1), Mem)

    scores = scores_out.reshape(NVID, T, 1)
    ssel_flat = ssel.reshape(NVID, 1)
    score_normal = ssel_flat[:BS]
    score_abnormal = ssel_flat[BS:]
    fs_flat = fs.reshape(NVID, NC)
    feat_select_normal = fs_flat[:BS].reshape(BS * NC)
    feat_select_abn = fs_flat[BS:].reshape(BS * NC)
    return (score_abnormal, score_normal, feat_select_abn, feat_select_normal,
            scores, s_loss.reshape(()), c_loss.reshape(()), u_loss.reshape(()))


# speedup vs baseline: 1.2884x; 1.2884x over previous
import jax
import jax.numpy as jnp
from jax.experimental import pallas as pl

BS = 8
NC = 10
T = 32
F = 2048
H = 2048
MEM = 60
NVID = 2 * BS
ROWS = NVID * NC * T
K = T // 16 + 1
VG = 4


def _mlp_kernel(x_ref, wa1_ref, ba1_ref, wa2_ref, ba2_ref, w1_ref, b1_ref,
                w2_ref, b2_ref, w3_ref, b3_ref, out_ref, score_ref, fmag_ref):
    x = x_ref[...]
    h = jnp.maximum(
        jax.lax.dot(x, wa1_ref[...], preferred_element_type=jnp.float32)
        + ba1_ref[...], 0.0)
    out = x + jax.lax.dot(h, wa2_ref[...],
                          preferred_element_type=jnp.float32) + ba2_ref[...]
    out_ref[...] = out
    fmag_ref[...] = jnp.sqrt(jnp.sum(out * out, axis=1, keepdims=True))
    s = jnp.maximum(
        jax.lax.dot(out, w1_ref[...], preferred_element_type=jnp.float32)
        + b1_ref[...], 0.0)
    s = jnp.maximum(
        jax.lax.dot(s, w2_ref[...], preferred_element_type=jnp.float32)
        + b2_ref[...], 0.0)
    score_ref[...] = jax.nn.sigmoid(
        jax.lax.dot(s, w3_ref[...], preferred_element_type=jnp.float32)
        + b3_ref[...])


def _select_kernel(feat_ref, score_ref, fmag_ref, mask_ref, mem_ref,
                   scores_out_ref, ssel_ref, fs_ref, sacc_ref, cacc_ref,
                   u_ref):
    i = pl.program_id(0)

    score_mean = jnp.mean(score_ref[...], axis=1)
    scores_out_ref[...] = score_mean[:, None, :]
    fmag_mean = jnp.mean(fmag_ref[...], axis=1)
    mask_mean = jnp.mean(mask_ref[...], axis=1)
    drop = fmag_mean * mask_mean

    iota = jax.lax.broadcasted_iota(jnp.int32, (VG, T), 1)
    v = drop
    wsel = jnp.zeros((VG, T), jnp.float32)
    for _ in range(K):
        m = jnp.max(v, axis=1, keepdims=True)
        idx = jnp.min(jnp.where(v >= m, iota, T), axis=1, keepdims=True)
        hit = iota == idx
        wsel = wsel + jnp.where(hit, 1.0, 0.0)
        v = jnp.where(hit, -jnp.inf, v)

    ssel_ref[...] = (jnp.sum(score_mean * wsel, axis=1, keepdims=True)
                     / K)[:, :, None]

    feats = feat_ref[...]
    sel = jnp.sum(feats * (wsel / K)[:, None, :, None], axis=2)
    fs_ref[...] = jnp.sqrt(jnp.sum(sel * sel, axis=2))[:, None, :]

    mem = mem_ref[...]
    sel2 = sel.reshape(VG * NC, F)
    logits = jax.lax.dot_general(sel2, mem, (((1,), (1,)), ((), ())),
                                 preferred_element_type=jnp.float32)
    att = jax.nn.softmax(logits, axis=-1)
    recon = jax.lax.dot(att, mem, preferred_element_type=jnp.float32)
    diff = recon - sel2
    s_contrib = jnp.sum(diff * diff).reshape(1, 1)
    c_contrib = jnp.sum(att * jnp.log(att + 1e-8)).reshape(1, 1)

    @pl.when(i == 0)
    def _():
        sacc_ref[...] = jnp.zeros((1, 1), jnp.float32)
        cacc_ref[...] = jnp.zeros((1, 1), jnp.float32)
        mn = mem / (jnp.sqrt(jnp.sum(mem * mem, axis=1, keepdims=True)) + 1e-8)
        g = jax.lax.dot_general(mn, mn, (((1,), (1,)), ((), ())),
                                preferred_element_type=jnp.float32)
        r = jax.lax.broadcasted_iota(jnp.int32, (MEM, MEM), 0)
        c = jax.lax.broadcasted_iota(jnp.int32, (MEM, MEM), 1)
        gd = g - jnp.where(r == c, 1.0, 0.0)
        u_ref[...] = (jnp.sum(gd * gd) / (MEM * MEM)).reshape(1, 1)

    sacc_ref[...] += s_contrib
    cacc_ref[...] += c_contrib

    @pl.when(i == NVID // VG - 1)
    def _():
        sacc_ref[...] = sacc_ref[...] / (NVID * NC * F)
        cacc_ref[...] = -cacc_ref[...] / (NVID * NC)


@jax.jit
def kernel(inputs, mask, Wa1, ba1, Wa2, ba2, W1, b1, W2, b2, W3, b3, Mem):
    x = inputs.reshape(ROWS, F)
    R = 512
    out, score_rows, fmag_rows = pl.pallas_call(
        _mlp_kernel,
        grid=(ROWS // R,),
        in_specs=[
            pl.BlockSpec((R, F), lambda i: (i, 0)),
            pl.BlockSpec((F, H), lambda i: (0, 0)),
            pl.BlockSpec((1, H), lambda i: (0, 0)),
            pl.BlockSpec((H, F), lambda i: (0, 0)),
            pl.BlockSpec((1, F), lambda i: (0, 0)),
            pl.BlockSpec((F, 512), lambda i: (0, 0)),
            pl.BlockSpec((1, 512), lambda i: (0, 0)),
            pl.BlockSpec((512, 128), lambda i: (0, 0)),
            pl.BlockSpec((1, 128), lambda i: (0, 0)),
            pl.BlockSpec((128, 1), lambda i: (0, 0)),
            pl.BlockSpec((1, 1), lambda i: (0, 0)),
        ],
        out_specs=[
            pl.BlockSpec((R, F), lambda i: (i, 0)),
            pl.BlockSpec((R, 1), lambda i: (i, 0)),
            pl.BlockSpec((R, 1), lambda i: (i, 0)),
        ],
        out_shape=[
            jax.ShapeDtypeStruct((ROWS, F), jnp.float32),
            jax.ShapeDtypeStruct((ROWS, 1), jnp.float32),
            jax.ShapeDtypeStruct((ROWS, 1), jnp.float32),
        ],
    )(x, Wa1, ba1.reshape(1, H), Wa2, ba2.reshape(1, F), W1,
      b1.reshape(1, 512), W2, b2.reshape(1, 128), W3, b3.reshape(1, 1))

    feats4 = out.reshape(NVID, NC, T, F)
    scores3 = score_rows.reshape(NVID, NC, T)
    fmag3 = fmag_rows.reshape(NVID, NC, T)
    mask3 = mask.reshape(NVID, NC, T)

    scores_out, ssel, fs, s_loss, c_loss, u_loss = pl.pallas_call(
        _select_kernel,
        grid=(NVID // VG,),
        in_specs=[
            pl.BlockSpec((VG, NC, T, F), lambda i: (i, 0, 0, 0)),
            pl.BlockSpec((VG, NC, T), lambda i: (i, 0, 0)),
            pl.BlockSpec((VG, NC, T), lambda i: (i, 0, 0)),
            pl.BlockSpec((VG, NC, T), lambda i: (i, 0, 0)),
            pl.BlockSpec((MEM, F), lambda i: (0, 0)),
        ],
        out_specs=[
            pl.BlockSpec((VG, 1, T), lambda i: (i, 0, 0)),
            pl.BlockSpec((VG, 1, 1), lambda i: (i, 0, 0)),
            pl.BlockSpec((VG, 1, NC), lambda i: (i, 0, 0)),
            pl.BlockSpec((1, 1), lambda i: (0, 0)),
            pl.BlockSpec((1, 1), lambda i: (0, 0)),
            pl.BlockSpec((1, 1), lambda i: (0, 0)),
        ],
        out_shape=[
            jax.ShapeDtypeStruct((NVID, 1, T), jnp.float32),
            jax.ShapeDtypeStruct((NVID, 1, 1), jnp.float32),
            jax.ShapeDtypeStruct((NVID, 1, NC), jnp.float32),
            jax.ShapeDtypeStruct((1, 1), jnp.float32),
            jax.ShapeDtypeStruct((1, 1), jnp.float32),
            jax.ShapeDtypeStruct((1, 1), jnp.float32),
        ],
    )(feats4, scores3, fmag3, mask3, Mem)

    scores = scores_out.reshape(NVID, T, 1)
    ssel_flat = ssel.reshape(NVID, 1)
    score_normal = ssel_flat[:BS]
    score_abnormal = ssel_flat[BS:]
    fs_flat = fs.reshape(NVID, NC)
    feat_select_normal = fs_flat[:BS].reshape(BS * NC)
    feat_select_abn = fs_flat[BS:].reshape(BS * NC)
    return (score_abnormal, score_normal, feat_select_abn, feat_select_normal,
            scores, s_loss.reshape(()), c_loss.reshape(()), u_loss.reshape(()))
